# Initial kernel scaffold; baseline (speedup 1.0000x reference)
import jax
import jax.numpy as jnp
from jax.experimental import pallas as pl


def kernel(x, edge_index, edge_attr, batch, w1, b1, w2, b2, lw1, lb1, lw2, lb2):
    raise NotImplementedError("write your pallas kernel here")


# speedup vs baseline: 1.0001x; 1.0001x over previous
import jax
import jax.numpy as jnp
from jax.experimental import pallas as pl
from jax.experimental.pallas import tpu as pltpu

N = 100000
E = 1600000
HID = 32
INPUT_SIZE = 10000
OUT_SIZE = 33
K = 5
NB = 10


def _head_body(h_ref, lw1_ref, lb1_ref, lw2_ref, lb2_ref, out_ref):
    h = h_ref[...]
    a = jnp.dot(h, lw1_ref[...], preferred_element_type=jnp.float32) + lb1_ref[...]
    a = jnp.maximum(a, 0.0)
    o = jnp.dot(a, lw2_ref[...], preferred_element_type=jnp.float32) + lb2_ref[...]
    out_ref[...] = jnp.clip(o, 0.0, 110.0)


def _mlp_head(pooled, lw1, lb1, lw2, lb2):
    return pl.pallas_call(
        _head_body,
        out_shape=jax.ShapeDtypeStruct((NB, OUT_SIZE), jnp.float32),
    )(pooled, lw1, lb1.reshape(1, 1000), lw2, lb2.reshape(1, OUT_SIZE))


def _cheb_conv(x, row, col, norm, W, b):
    def prop(t):
        return jax.ops.segment_sum(norm[:, None] * t[row], col, num_segments=N)
    Tx0 = x
    out = Tx0 @ W[0]
    Tx1 = prop(Tx0)
    out = out + Tx1 @ W[1]
    for k in range(2, K):
        Tx2 = 2.0 * prop(Tx1) - Tx0
        out = out + Tx2 @ W[k]
        Tx0, Tx1 = Tx1, Tx2
    return out + b


def kernel(x, edge_index, edge_attr, batch, w1, b1, w2, b2, lw1, lb1, lw2, lb2):
    row, col = edge_index[0], edge_index[1]
    ew = jnp.where(row != col, 1.0, 0.0).astype(x.dtype)
    deg = jax.ops.segment_sum(ew, row, num_segments=N)
    dis = jnp.where(deg > 0, deg ** -0.5, 0.0)
    norm = -dis[row] * ew * dis[col]

    h = jax.nn.relu(_cheb_conv(x, row, col, norm, w1, b1))
    h = jax.nn.relu(_cheb_conv(h, row, col, norm, w2, b2))
    pooled = h.mean(axis=1).reshape(NB, INPUT_SIZE)
    out = _mlp_head(pooled, lw1, lb1, lw2, lb2)
    return jnp.squeeze(out)


# speedup vs baseline: 26.3957x; 26.3923x over previous
import functools

import jax
import jax.numpy as jnp
from jax import lax
from jax.experimental import pallas as pl
from jax.experimental.pallas import tpu as pltpu
from jax.experimental.pallas import tpu_sc as plsc

N = 100000
E = 1600000
HID = 32
INPUT_SIZE = 10000
OUT_SIZE = 33
K = 5
NB = 10

NPAD = 100352
RPS = NPAD // 16
WSTG = RPS // 16
CH = 128
WCHUNK = 784
SCHUNK = 392
EPAD = 16 * WCHUNK * CH

_MESH = plsc.VectorSubcoreMesh(core_axis_name="c", subcore_axis_name="s")
_SC_PARAMS = pltpu.CompilerParams(use_tc_tiling_on_sc=False,
                                 internal_scratch_in_bytes=0)


def _prop_wide_body(tin, rowp, colp, out, idxr, idxc, buf, stage, acc,
                    sem_g, sem_s):
    c = lax.axis_index("c")
    s = lax.axis_index("s")
    w = c * 16 + s

    def zrow(i, carry):
        stage[i, :] = jnp.zeros((16,), jnp.float32)
        return carry

    lax.fori_loop(0, WSTG, zrow, 0)
    for g in range(16):
        pltpu.sync_copy(stage, acc.at[pl.ds(s * RPS + g * WSTG, WSTG), :])
    plsc.subcore_barrier()

    def chunk(g, carry):
        pltpu.sync_copy(rowp.at[w, pl.ds(g * 8, 8), :], idxr)
        pltpu.sync_copy(colp.at[s, pl.ds(g * 8, 8), :], idxc)
        gathers = [
            pltpu.async_copy(tin.at[idxr.at[j]], buf.at[j], sem_g)
            for j in range(8)
        ]
        for d in gathers:
            d.wait()
        scatters = [
            pltpu.async_copy(buf.at[j], acc.at[idxc.at[j]], sem_s, add=True)
            for j in range(8)
        ]
        for d in scatters:
            d.wait()
        return carry

    lax.fori_loop(0, WCHUNK // 8, chunk, 0)
    plsc.subcore_barrier()

    for g in range(16):
        pltpu.sync_copy(acc.at[pl.ds(s * RPS + g * WSTG, WSTG), :], stage)
        pltpu.sync_copy(stage, out.at[c, pl.ds(s * RPS + g * WSTG, WSTG), :])


_prop_wide = functools.partial(
    pl.kernel,
    out_type=jax.ShapeDtypeStruct((2, NPAD, 16), jnp.float32),
    mesh=_MESH,
    compiler_params=_SC_PARAMS,
    scratch_types=[
        pltpu.VMEM((8, CH), jnp.int32),
        pltpu.VMEM((8, CH), jnp.int32),
        pltpu.VMEM((8, CH, 16), jnp.float32),
        pltpu.VMEM((WSTG, 16), jnp.float32),
        pltpu.VMEM_SHARED((NPAD, 16), jnp.float32),
        pltpu.SemaphoreType.DMA,
        pltpu.SemaphoreType.DMA,
    ],
)(_prop_wide_body)


def _prop_scal_body(tin, rowg, scat, out, idxr, idxc, buf, stage, acc,
                    sem_g, sem_s):
    c = lax.axis_index("c")
    s = lax.axis_index("s")
    w = c * 16 + s

    def zrow(i, carry):
        stage[pl.ds(i * 16, 16)] = jnp.zeros((16,), jnp.float32)
        return carry

    lax.fori_loop(0, RPS // 16, zrow, 0)
    pltpu.sync_copy(stage, acc.at[pl.ds(s * RPS, RPS)])
    plsc.subcore_barrier()

    def chunk(g, carry):
        pltpu.sync_copy(rowg.at[w, pl.ds(g * 8, 8), :], idxr)
        pltpu.sync_copy(scat.at[w, pl.ds(g * 8, 8), :], idxc)
        gathers = [
            pltpu.async_copy(tin.at[idxr.at[j]], buf.at[j], sem_g)
            for j in range(8)
        ]
        for d in gathers:
            d.wait()
        scatters = [
            pltpu.async_copy(buf.at[j], acc.at[idxc.at[j]], sem_s, add=True)
            for j in range(8)
        ]
        for d in scatters:
            d.wait()
        return carry

    lax.fori_loop(0, SCHUNK // 8, chunk, 0)
    plsc.subcore_barrier()

    pltpu.sync_copy(acc.at[pl.ds(s * RPS, RPS)], stage)
    pltpu.sync_copy(stage, out.at[c, pl.ds(s * RPS, RPS)])


_prop_scal = functools.partial(
    pl.kernel,
    out_type=jax.ShapeDtypeStruct((2, NPAD), jnp.float32),
    mesh=_MESH,
    compiler_params=_SC_PARAMS,
    scratch_types=[
        pltpu.VMEM((8, CH), jnp.int32),
        pltpu.VMEM((8, CH), jnp.int32),
        pltpu.VMEM((8, CH), jnp.float32),
        pltpu.VMEM((RPS,), jnp.float32),
        pltpu.VMEM_SHARED((NPAD,), jnp.float32),
        pltpu.SemaphoreType.DMA,
        pltpu.SemaphoreType.DMA,
    ],
)(_prop_scal_body)


def _head_body(h_ref, lw1_ref, lb1_ref, lw2_ref, lb2_ref, out_ref):
    h = h_ref[...]
    a = jnp.dot(h, lw1_ref[...], preferred_element_type=jnp.float32) + lb1_ref[...]
    a = jnp.maximum(a, 0.0)
    o = jnp.dot(a, lw2_ref[...], preferred_element_type=jnp.float32) + lb2_ref[...]
    out_ref[...] = jnp.clip(o, 0.0, 110.0)


def _mlp_head(pooled, lw1, lb1, lw2, lb2):
    return pl.pallas_call(
        _head_body,
        out_shape=jax.ShapeDtypeStruct((NB, OUT_SIZE), jnp.float32),
    )(pooled, lw1, lb1.reshape(1, 1000), lw2, lb2.reshape(1, OUT_SIZE))


def _pad_e(a, fill):
    return jnp.concatenate([a, jnp.full((EPAD - E,), fill, a.dtype)])


def kernel(x, edge_index, edge_attr, batch, w1, b1, w2, b2, lw1, lb1, lw2, lb2):
    row = edge_index[0]
    col = edge_index[1]
    nonself = row != col
    rowz = jnp.where(nonself, row, N)

    rp = _pad_e(rowz, N)
    cp = _pad_e(col, N)
    dp = _pad_e(row, N)

    rowg_s = rp.reshape(32, SCHUNK, CH)
    col_s = cp.reshape(32, SCHUNK, CH)
    deg_s = dp.reshape(32, SCHUNK, CH)
    rowp_w = jnp.stack([rp, rp + NPAD]).reshape(32, WCHUNK, CH)
    col_w = cp.reshape(16, WCHUNK, CH)

    zpad1 = jnp.zeros((NPAD - N,), jnp.float32)
    ones_t = jnp.concatenate([jnp.ones((N,), jnp.float32), zpad1])
    deg = _prop_scal(ones_t, rowg_s, deg_s).sum(axis=0)[:N]
    dis = jnp.where(deg > 0, lax.rsqrt(deg), 0.0)

    def prop1(t):
        tin = jnp.concatenate([dis * t, zpad1])
        p = _prop_scal(tin, rowg_s, col_s)
        return -dis * (p[0, :N] + p[1, :N])

    t0 = x[:, 0]
    txs = [t0, prop1(t0)]
    for _ in range(2, K):
        txs.append(2.0 * prop1(txs[-1]) - txs[-2])
    h = jax.nn.relu(jnp.stack(txs, axis=1) @ w1[:, 0, :] + b1)

    def prop2(t):
        sc = dis[:, None] * t
        zpad16 = jnp.zeros((NPAD - N, 16), jnp.float32)
        tin = jnp.concatenate([sc[:, :16], zpad16, sc[:, 16:], zpad16], axis=0)
        y = _prop_wide(tin, rowp_w, col_w)
        return -dis[:, None] * jnp.concatenate([y[0, :N, :], y[1, :N, :]], axis=1)

    tx0 = h
    out = tx0 @ w2[0]
    tx1 = prop2(tx0)
    out = out + tx1 @ w2[1]
    for k in range(2, K):
        tx2 = 2.0 * prop2(tx1) - tx0
        out = out + tx2 @ w2[k]
        tx0, tx1 = tx1, tx2
    h2 = jax.nn.relu(out + b2)

    pooled = h2.mean(axis=1).reshape(NB, INPUT_SIZE)
    return jnp.squeeze(_mlp_head(pooled, lw1, lb1, lw2, lb2))


# speedup vs baseline: 30.1072x; 1.1406x over previous
import functools

import jax
import jax.numpy as jnp
from jax import lax
from jax.experimental import pallas as pl
from jax.experimental.pallas import tpu as pltpu
from jax.experimental.pallas import tpu_sc as plsc

N = 100000
E = 1600000
HID = 32
INPUT_SIZE = 10000
OUT_SIZE = 33
K = 5
NB = 10

NPAD = 100352
RPS = NPAD // 16
WSTG = RPS // 16
CH = 128
WCHUNK = 784
SCHUNK = 392
EPAD = 16 * WCHUNK * CH

_MESH = plsc.VectorSubcoreMesh(core_axis_name="c", subcore_axis_name="s")
_SC_PARAMS = pltpu.CompilerParams(use_tc_tiling_on_sc=False,
                                 internal_scratch_in_bytes=0)


def _prop_wide_body(tin, rowp, colp, out, idxr, idxc, buf, stage, acc,
                    sem_g0, sem_g1, sem_s0, sem_s1):
    c = lax.axis_index("c")
    s = lax.axis_index("s")
    w = c * 16 + s

    def zrow(i, carry):
        stage[i, :] = jnp.zeros((16,), jnp.float32)
        return carry

    lax.fori_loop(0, WSTG, zrow, 0)
    for g in range(16):
        pltpu.sync_copy(stage, acc.at[pl.ds(s * RPS + g * WSTG, WSTG), :])
    plsc.subcore_barrier()

    def load_idx(slot, g):
        pltpu.sync_copy(rowp.at[w, pl.ds(g * 4, 4), :], idxr.at[slot])
        pltpu.sync_copy(colp.at[s, pl.ds(g * 4, 4), :], idxc.at[slot])

    NP = WCHUNK // 8
    load_idx(0, 0)
    for j in range(4):
        pltpu.async_copy(tin.at[idxr.at[0, j]], buf.at[0, j], sem_g0)

    def pair(t, carry):
        load_idx(1, 2 * t + 1)
        gath1 = [
            pltpu.async_copy(tin.at[idxr.at[1, j]], buf.at[1, j], sem_g1)
            for j in range(4)
        ]
        for j in range(4):
            pltpu.make_async_copy(tin.at[pl.ds(0, CH), :], buf.at[0, j],
                                  sem_g0).wait()
        scat0 = [
            pltpu.async_copy(buf.at[0, j], acc.at[idxc.at[0, j]], sem_s0,
                             add=True)
            for j in range(4)
        ]
        for d in scat0:
            d.wait()

        @pl.when(t + 1 < NP)
        def _prefetch():
            load_idx(0, 2 * t + 2)
            for j in range(4):
                pltpu.async_copy(tin.at[idxr.at[0, j]], buf.at[0, j], sem_g0)

        for d in gath1:
            d.wait()
        scat1 = [
            pltpu.async_copy(buf.at[1, j], acc.at[idxc.at[1, j]], sem_s1,
                             add=True)
            for j in range(4)
        ]
        for d in scat1:
            d.wait()
        return carry

    lax.fori_loop(0, NP, pair, 0)
    plsc.subcore_barrier()

    for g in range(16):
        pltpu.sync_copy(acc.at[pl.ds(s * RPS + g * WSTG, WSTG), :], stage)
        pltpu.sync_copy(stage, out.at[c, pl.ds(s * RPS + g * WSTG, WSTG), :])


_prop_wide = functools.partial(
    pl.kernel,
    out_type=jax.ShapeDtypeStruct((2, NPAD, 16), jnp.float32),
    mesh=_MESH,
    compiler_params=_SC_PARAMS,
    scratch_types=[
        pltpu.VMEM((2, 4, CH), jnp.int32),
        pltpu.VMEM((2, 4, CH), jnp.int32),
        pltpu.VMEM((2, 4, CH, 16), jnp.float32),
        pltpu.VMEM((WSTG, 16), jnp.float32),
        pltpu.VMEM_SHARED((NPAD, 16), jnp.float32),
        pltpu.SemaphoreType.DMA,
        pltpu.SemaphoreType.DMA,
        pltpu.SemaphoreType.DMA,
        pltpu.SemaphoreType.DMA,
    ],
)(_prop_wide_body)


def _prop_scal_body(tin, rowg, scat, out, idxr, idxc, buf, stage, acc,
                    sem_g0, sem_g1, sem_s0, sem_s1):
    c = lax.axis_index("c")
    s = lax.axis_index("s")
    w = c * 16 + s

    def zrow(i, carry):
        stage[pl.ds(i * 16, 16)] = jnp.zeros((16,), jnp.float32)
        return carry

    lax.fori_loop(0, RPS // 16, zrow, 0)
    pltpu.sync_copy(stage, acc.at[pl.ds(s * RPS, RPS)])
    plsc.subcore_barrier()

    def load_idx(slot, g):
        pltpu.sync_copy(rowg.at[w, pl.ds(g * 4, 4), :], idxr.at[slot])
        pltpu.sync_copy(scat.at[w, pl.ds(g * 4, 4), :], idxc.at[slot])

    NP = SCHUNK // 8
    load_idx(0, 0)
    for j in range(4):
        pltpu.async_copy(tin.at[idxr.at[0, j]], buf.at[0, j], sem_g0)

    def pair(t, carry):
        load_idx(1, 2 * t + 1)
        gath1 = [
            pltpu.async_copy(tin.at[idxr.at[1, j]], buf.at[1, j], sem_g1)
            for j in range(4)
        ]
        for j in range(4):
            pltpu.make_async_copy(tin.at[pl.ds(0, CH)], buf.at[0, j],
                                  sem_g0).wait()
        scat0 = [
            pltpu.async_copy(buf.at[0, j], acc.at[idxc.at[0, j]], sem_s0,
                             add=True)
            for j in range(4)
        ]
        for d in scat0:
            d.wait()

        @pl.when(t + 1 < NP)
        def _prefetch():
            load_idx(0, 2 * t + 2)
            for j in range(4):
                pltpu.async_copy(tin.at[idxr.at[0, j]], buf.at[0, j], sem_g0)

        for d in gath1:
            d.wait()
        scat1 = [
            pltpu.async_copy(buf.at[1, j], acc.at[idxc.at[1, j]], sem_s1,
                             add=True)
            for j in range(4)
        ]
        for d in scat1:
            d.wait()
        return carry

    lax.fori_loop(0, NP, pair, 0)
    plsc.subcore_barrier()

    pltpu.sync_copy(acc.at[pl.ds(s * RPS, RPS)], stage)
    pltpu.sync_copy(stage, out.at[c, pl.ds(s * RPS, RPS)])


_prop_scal = functools.partial(
    pl.kernel,
    out_type=jax.ShapeDtypeStruct((2, NPAD), jnp.float32),
    mesh=_MESH,
    compiler_params=_SC_PARAMS,
    scratch_types=[
        pltpu.VMEM((2, 4, CH), jnp.int32),
        pltpu.VMEM((2, 4, CH), jnp.int32),
        pltpu.VMEM((2, 4, CH), jnp.float32),
        pltpu.VMEM((RPS,), jnp.float32),
        pltpu.VMEM_SHARED((NPAD,), jnp.float32),
        pltpu.SemaphoreType.DMA,
        pltpu.SemaphoreType.DMA,
        pltpu.SemaphoreType.DMA,
        pltpu.SemaphoreType.DMA,
    ],
)(_prop_scal_body)


def _head_body(h_ref, lw1_ref, lb1_ref, lw2_ref, lb2_ref, out_ref):
    h = h_ref[...]
    a = jnp.dot(h, lw1_ref[...], preferred_element_type=jnp.float32) + lb1_ref[...]
    a = jnp.maximum(a, 0.0)
    o = jnp.dot(a, lw2_ref[...], preferred_element_type=jnp.float32) + lb2_ref[...]
    out_ref[...] = jnp.clip(o, 0.0, 110.0)


def _mlp_head(pooled, lw1, lb1, lw2, lb2):
    return pl.pallas_call(
        _head_body,
        out_shape=jax.ShapeDtypeStruct((NB, OUT_SIZE), jnp.float32),
    )(pooled, lw1, lb1.reshape(1, 1000), lw2, lb2.reshape(1, OUT_SIZE))


def _pad_e(a, fill):
    return jnp.concatenate([a, jnp.full((EPAD - E,), fill, a.dtype)])


def kernel(x, edge_index, edge_attr, batch, w1, b1, w2, b2, lw1, lb1, lw2, lb2):
    row = edge_index[0]
    col = edge_index[1]
    nonself = row != col
    rowz = jnp.where(nonself, row, N)

    rp = _pad_e(rowz, N)
    cp = _pad_e(col, N)
    dp = _pad_e(row, N)

    rowg_s = rp.reshape(32, SCHUNK, CH)
    col_s = cp.reshape(32, SCHUNK, CH)
    deg_s = dp.reshape(32, SCHUNK, CH)
    rowp_w = jnp.stack([rp, rp + NPAD]).reshape(32, WCHUNK, CH)
    col_w = cp.reshape(16, WCHUNK, CH)

    zpad1 = jnp.zeros((NPAD - N,), jnp.float32)
    ones_t = jnp.concatenate([jnp.ones((N,), jnp.float32), zpad1])
    deg = _prop_scal(ones_t, rowg_s, deg_s).sum(axis=0)[:N]
    dis = jnp.where(deg > 0, lax.rsqrt(deg), 0.0)

    def prop1(t):
        tin = jnp.concatenate([dis * t, zpad1])
        p = _prop_scal(tin, rowg_s, col_s)
        return -dis * (p[0, :N] + p[1, :N])

    t0 = x[:, 0]
    txs = [t0, prop1(t0)]
    for _ in range(2, K):
        txs.append(2.0 * prop1(txs[-1]) - txs[-2])
    h = jax.nn.relu(jnp.stack(txs, axis=1) @ w1[:, 0, :] + b1)

    def prop2(t):
        sc = dis[:, None] * t
        zpad16 = jnp.zeros((NPAD - N, 16), jnp.float32)
        tin = jnp.concatenate([sc[:, :16], zpad16, sc[:, 16:], zpad16], axis=0)
        y = _prop_wide(tin, rowp_w, col_w)
        return -dis[:, None] * jnp.concatenate([y[0, :N, :], y[1, :N, :]], axis=1)

    tx0 = h
    out = tx0 @ w2[0]
    tx1 = prop2(tx0)
    out = out + tx1 @ w2[1]
    for k in range(2, K):
        tx2 = 2.0 * prop2(tx1) - tx0
        out = out + tx2 @ w2[k]
        tx0, tx1 = tx1, tx2
    h2 = jax.nn.relu(out + b2)

    pooled = h2.mean(axis=1).reshape(NB, INPUT_SIZE)
    return jnp.squeeze(_mlp_head(pooled, lw1, lb1, lw2, lb2))
